# Initial kernel scaffold; baseline (speedup 1.0000x reference)
import jax
import jax.numpy as jnp
from jax.experimental import pallas as pl


def kernel(hidden_states, weight, e_score_correction_bias):
    raise NotImplementedError("write your pallas kernel here")


# speedup vs baseline: 1.1842x; 1.1842x over previous
import functools

import jax
import jax.numpy as jnp
from jax import lax
from jax.experimental import pallas as pl
from jax.experimental.pallas import tpu as pltpu
from jax.experimental.pallas import tpu_sc as plsc

HIDDEN = 4096
EXPERTS = 64
TOPK = 8
T_TOTAL = 4 * 8192

TC_BLK = 512

NUM_CORES = 2
NUM_SUBCORES = 16
NW = NUM_CORES * NUM_SUBCORES
TPW = T_TOTAL // NW
LANES = 16


def _tc_scores_body(h_ref, w_ref, out_ref):
    logits = lax.dot_general(
        h_ref[...], w_ref[...],
        dimension_numbers=(((1,), (1,)), ((), ())),
        preferred_element_type=jnp.float32,
    )
    out_ref[...] = jax.nn.sigmoid(logits)


def _tc_scores(h, weight):
    return pl.pallas_call(
        _tc_scores_body,
        grid=(T_TOTAL // TC_BLK,),
        in_specs=[
            pl.BlockSpec((TC_BLK, HIDDEN), lambda i: (i, 0)),
            pl.BlockSpec((EXPERTS, HIDDEN), lambda i: (0, 0)),
        ],
        out_specs=pl.BlockSpec((TC_BLK, EXPERTS), lambda i: (i, 0)),
        out_shape=jax.ShapeDtypeStruct((T_TOTAL, EXPERTS), jnp.float32),
    )(h, weight)


def _merge_top16(ak, av, bk, bv):
    rk = lax.rev(bk, (0,))
    rv = lax.rev(bv, (0,))
    c = ak >= rk
    mk = jnp.where(c, ak, rk)
    mv = jnp.where(c, av, rv)
    return plsc.sort_key_val(mk, mv, descending=True)


SC_CHUNK = 512


def _sc_topk_body(scores_hbm, bias_hbm, idx_hbm, w_hbm,
                  scores_v, bias_v, idx_v, w_v):
    wid = lax.axis_index("s") * NUM_CORES + lax.axis_index("c")
    base = wid * TPW
    pltpu.sync_copy(bias_hbm, bias_v)

    iota = lax.iota(jnp.int32, LANES)
    mask8 = iota < TOPK
    col = jnp.where(mask8, iota, 0)
    bias_c = [bias_v[pl.ds(16 * i, 16)] for i in range(4)]
    idx_c = [iota + 16 * i for i in range(4)]

    def chunk_body(c, carry):
        cbase = base + c * SC_CHUNK
        pltpu.sync_copy(scores_hbm.at[pl.ds(cbase, SC_CHUNK)], scores_v)

        def body(t, carry):
            srt = []
            for i in range(4):
                s = scores_v[t, pl.ds(16 * i, 16)]
                srt.append(plsc.sort_key_val(s + bias_c[i], idx_c[i],
                                             descending=True))
            k01, v01 = _merge_top16(*srt[0], *srt[1])
            k23, v23 = _merge_top16(*srt[2], *srt[3])
            _, fi = _merge_top16(k01, v01, k23, v23)
            t_s = jnp.full((LANES,), t, jnp.int32)
            w = plsc.load_gather(scores_v, [t_s, fi])
            wz = jnp.where(mask8, w, 0.0)
            wn = wz / (jnp.sum(wz) + 1e-20)
            plsc.store_scatter(idx_v, [t_s, col], fi, mask=mask8)
            plsc.store_scatter(w_v, [t_s, col], wn, mask=mask8)
            return carry

        lax.fori_loop(0, SC_CHUNK, body, 0, unroll=4)
        pltpu.sync_copy(idx_v, idx_hbm.at[pl.ds(cbase, SC_CHUNK)])
        pltpu.sync_copy(w_v, w_hbm.at[pl.ds(cbase, SC_CHUNK)])
        return carry

    lax.fori_loop(0, TPW // SC_CHUNK, chunk_body, 0)


_sc_topk = functools.partial(
    pl.kernel,
    out_type=(
        jax.ShapeDtypeStruct((T_TOTAL, TOPK), jnp.int32),
        jax.ShapeDtypeStruct((T_TOTAL, TOPK), jnp.float32),
    ),
    mesh=plsc.VectorSubcoreMesh(core_axis_name="c", subcore_axis_name="s",
                                num_cores=NUM_CORES,
                                num_subcores=NUM_SUBCORES),
    scratch_types=[
        pltpu.VMEM((SC_CHUNK, EXPERTS), jnp.float32),
        pltpu.VMEM((EXPERTS,), jnp.float32),
        pltpu.VMEM((SC_CHUNK, TOPK), jnp.int32),
        pltpu.VMEM((SC_CHUNK, TOPK), jnp.float32),
    ],
    compiler_params=pltpu.CompilerParams(needs_layout_passes=False,
                                         use_tc_tiling_on_sc=False),
)(_sc_topk_body)


def kernel(hidden_states, weight, e_score_correction_bias):
    h = hidden_states.reshape(T_TOTAL, HIDDEN)
    scores = _tc_scores(h, weight)
    topk_indices, topk_weights = _sc_topk(scores, e_score_correction_bias)
    return topk_indices, topk_weights
